# baseline (speedup 1.0000x reference)
import functools

import jax
import jax.numpy as jnp
from jax import lax
from jax.experimental import pallas as pl
from jax.experimental.pallas import tpu as pltpu
from jax.experimental.pallas import tpu_sc as plsc

_NC = 2
_NS = 16
_NW = _NC * _NS

_mesh = plsc.VectorSubcoreMesh(core_axis_name="c", subcore_axis_name="s")


@functools.lru_cache(maxsize=None)
def _make_gather(h_tot, d):
    assert d == 32
    nh = h_tot

    @functools.partial(
        pl.kernel,
        mesh=_mesh,
        compiler_params=pltpu.CompilerParams(use_tc_tiling_on_sc=False, needs_layout_passes=False),
        out_type=jax.ShapeDtypeStruct((nh, 4, _NW, 1024), jnp.float32),
        scratch_types=[
            pltpu.VMEM((nh, 128), jnp.int32),
            pltpu.VMEM((2, 128, 32), jnp.float32),
            pltpu.VMEM((4096,), jnp.float32),
            pltpu.VMEM((4096,), jnp.float32),
            pltpu.SemaphoreType.DMA,
            pltpu.SemaphoreType.DMA,
        ],
    )
    def gather(table_hbm, idx_hbm, out_hbm, idx_v, rows_v, cb0, cb1, gsem, osem):
        wid = lax.axis_index("s") * _NC + lax.axis_index("c")
        pltpu.sync_copy(idx_hbm.at[wid], idx_v)

        it = lax.iota(jnp.int32, 16)
        b_lo = it * 128
        b_hi = it * 128 + 2048

        def fire(h, slot):
            pltpu.async_copy(table_hbm.at[idx_v.at[h]], rows_v.at[slot], gsem)

        fire(0, 0)

        @pl.loop(0, nh, step=2)
        def _(h0):
            for s in range(2):
                h = h0 + s
                pltpu.make_async_copy(
                    table_hbm.at[idx_v.at[h]], rows_v.at[s], gsem
                ).wait()

                cb = (cb0, cb1)[s]
                cbo = (cb0, cb1)[1 - s]

                @pl.when(h >= 1)
                def _():
                    for o in range(4):
                        pltpu.make_async_copy(
                            cbo.at[pl.ds(o * 1024, 1024)],
                            out_hbm.at[0, o, wid],
                            osem,
                        ).wait()

                @pl.when(h + 1 < nh)
                def _():
                    fire(h + 1, 1 - s)

                for bl in range(128):
                    v0 = rows_v[s, bl, pl.ds(0, 16)]
                    plsc.store_scatter(cb, [b_lo + bl], v0)
                    v1 = rows_v[s, bl, pl.ds(16, 16)]
                    plsc.store_scatter(cb, [b_hi + bl], v1)

                for o in range(4):
                    pltpu.async_copy(
                        cb.at[pl.ds(o * 1024, 1024)],
                        out_hbm.at[h, o, wid],
                        osem,
                    )

        for o in range(4):
            pltpu.make_async_copy(
                (cb0, cb1)[(nh - 1) % 2].at[pl.ds(o * 1024, 1024)],
                out_hbm.at[0, o, wid],
                osem,
            ).wait()

    return gather


def kernel(X, weights):
    b, h = X.shape
    d = weights.shape[1]
    idx_t = jnp.transpose(
        jnp.reshape(X.astype(jnp.int32), (_NW, b // _NW, h)), (0, 2, 1)
    )
    of = _make_gather(h, d)(weights, idx_t)
    l5 = jnp.reshape(of, (h, 4, _NW, 8, b // _NW))
    return jnp.reshape(jnp.transpose(l5, (2, 4, 0, 1, 3)), (b, h, d))


# speedup vs baseline: 1.3979x; 1.3979x over previous
import functools

import jax
import jax.numpy as jnp
from jax import lax
from jax.experimental import pallas as pl
from jax.experimental.pallas import tpu as pltpu
from jax.experimental.pallas import tpu_sc as plsc

_NC = 2
_NS = 16
_NW = _NC * _NS

_mesh = plsc.VectorSubcoreMesh(core_axis_name="c", subcore_axis_name="s")


@functools.lru_cache(maxsize=None)
def _make_gather(h_tot, d):
    assert d == 32
    nh = h_tot

    @functools.partial(
        pl.kernel,
        mesh=_mesh,
        compiler_params=pltpu.CompilerParams(use_tc_tiling_on_sc=False, needs_layout_passes=False),
        out_type=jax.ShapeDtypeStruct((nh, 4, _NW, 8, 128), jnp.float32),
        scratch_types=[
            pltpu.VMEM((nh, 128), jnp.int32),
            pltpu.VMEM((2, 128, 32), jnp.float32),
            pltpu.VMEM((32, 129), jnp.float32),
            pltpu.VMEM((32, 129), jnp.float32),
            pltpu.SemaphoreType.DMA,
            pltpu.SemaphoreType.DMA,
        ],
    )
    def gather(table_hbm, idx_hbm, out_hbm, idx_v, rows_v, cb0, cb1, gsem, osem):
        wid = lax.axis_index("s") * _NC + lax.axis_index("c")
        pltpu.sync_copy(idx_hbm.at[wid], idx_v)

        it = lax.iota(jnp.int32, 16)
        w_lo = it
        w_hi = it + 16

        def fire(h, slot):
            pltpu.async_copy(table_hbm.at[idx_v.at[h]], rows_v.at[slot], gsem)

        fire(0, 0)

        @pl.loop(0, nh, step=2)
        def _(h0):
            for s in range(2):
                h = h0 + s
                pltpu.make_async_copy(
                    table_hbm.at[idx_v.at[h]], rows_v.at[s], gsem
                ).wait()

                cb = (cb0, cb1)[s]
                cbo = (cb0, cb1)[1 - s]

                @pl.when(h >= 1)
                def _():
                    for o in range(4):
                        pltpu.make_async_copy(
                            cbo.at[pl.ds(o * 8, 8), pl.ds(0, 128)],
                            out_hbm.at[0, o, wid],
                            osem,
                        ).wait()

                @pl.when(h + 1 < nh)
                def _():
                    fire(h + 1, 1 - s)

                for bl in range(128):
                    blv = it * 0 + bl
                    v0 = rows_v[s, bl, pl.ds(0, 16)]
                    plsc.store_scatter(cb, [w_lo, blv], v0)
                    v1 = rows_v[s, bl, pl.ds(16, 16)]
                    plsc.store_scatter(cb, [w_hi, blv], v1)

                for o in range(4):
                    pltpu.async_copy(
                        cb.at[pl.ds(o * 8, 8), pl.ds(0, 128)],
                        out_hbm.at[h, o, wid],
                        osem,
                    )

        for o in range(4):
            pltpu.make_async_copy(
                (cb0, cb1)[(nh - 1) % 2].at[pl.ds(o * 8, 8), pl.ds(0, 128)],
                out_hbm.at[0, o, wid],
                osem,
            ).wait()

    return gather


def kernel(X, weights):
    b, h = X.shape
    d = weights.shape[1]
    idx_t = jnp.transpose(
        jnp.reshape(X.astype(jnp.int32), (_NW, b // _NW, h)), (0, 2, 1)
    )
    l5 = _make_gather(h, d)(weights, idx_t)
    return jnp.reshape(jnp.transpose(l5, (2, 4, 0, 1, 3)), (b, h, d))
